# baseline (speedup 1.0000x reference)
import jax
import jax.numpy as jnp
from jax.experimental import pallas as pl
import jax.experimental.pallas.tpu as pltpu

_BM = 1024


def _prep_body(w1_ref, w2_ref, w1a_ref, w2b_ref):
    w1a_ref[...] = w1_ref[0:w1a_ref.shape[0], :].astype(jnp.bfloat16)
    w2b_ref[...] = w2_ref[...].astype(jnp.bfloat16)


def _mlp_body(emb_ref, small_ref, mask_ref,
              w1a_ref, w1b_ref, w2_ref, b1_ref, b2_ref, out_ref, h_ref):
    out = jnp.dot(h_ref[...], w2_ref[...], preferred_element_type=jnp.float32)
    out += b2_ref[...]
    out_ref[...] = out * mask_ref[...]
    x = emb_ref[...].astype(jnp.bfloat16)
    acc = jnp.dot(x, w1a_ref[...], preferred_element_type=jnp.float32)
    acc += jnp.dot(small_ref[...], w1b_ref[...],
                   preferred_element_type=jnp.float32)
    acc += b1_ref[...]
    h_ref[...] = jnp.maximum(acc, 0.0).astype(jnp.bfloat16)


def kernel(embeddings, visibility_scores, bbox_ltwh, keypoints_xyc,
           feats_masks, W1, b1, W2, b2):
    B, N, E = embeddings.shape
    M = B * N
    KPF = keypoints_xyc.shape[2] * keypoints_xyc.shape[3]
    F = W1.shape[1]
    T = W2.shape[1]
    S = F - E
    G = M // _BM

    emb = embeddings.reshape(G, _BM, E)
    small = jnp.concatenate(
        [visibility_scores.reshape(M, 1),
         bbox_ltwh.reshape(M, bbox_ltwh.shape[-1]),
         keypoints_xyc.reshape(M, KPF)],
        axis=-1).astype(jnp.bfloat16)
    maskf = feats_masks.reshape(M, 1).astype(jnp.float32)
    W1b = W1[E:].astype(jnp.bfloat16)
    b1r = b1.reshape(1, F)
    b2r = b2.reshape(1, T)

    W1a, W2b = pl.pallas_call(
        _prep_body,
        out_shape=(
            jax.ShapeDtypeStruct((E, F), jnp.bfloat16),
            jax.ShapeDtypeStruct((F, T), jnp.bfloat16),
        ),
    )(W1, W2)

    def _cur(i):
        return jnp.minimum(i, G - 1)

    def _prev(i):
        return jnp.maximum(i - 1, 0)

    out = pl.pallas_call(
        _mlp_body,
        grid=(G + 1,),
        in_specs=[
            pl.BlockSpec((None, _BM, E), lambda i: (_cur(i), 0, 0)),
            pl.BlockSpec((_BM, S), lambda i: (_cur(i), 0)),
            pl.BlockSpec((_BM, 1), lambda i: (_prev(i), 0)),
            pl.BlockSpec((E, F), lambda i: (0, 0)),
            pl.BlockSpec((S, F), lambda i: (0, 0)),
            pl.BlockSpec((F, T), lambda i: (0, 0)),
            pl.BlockSpec((1, F), lambda i: (0, 0)),
            pl.BlockSpec((1, T), lambda i: (0, 0)),
        ],
        out_specs=pl.BlockSpec((None, _BM, T), lambda i: (_prev(i), 0, 0)),
        out_shape=jax.ShapeDtypeStruct((G, _BM, T), jnp.float32),
        scratch_shapes=[pltpu.VMEM((_BM, F), jnp.bfloat16)],
    )(emb, small, maskf, W1a, W1b, W2b, b1r, b2r)
    return out.reshape(B, N, T)


# speedup vs baseline: 1.0258x; 1.0258x over previous
import jax
import jax.numpy as jnp
from jax.experimental import pallas as pl

_BM = 1024


def _prep_body(w1_ref, w2_ref, w1a_ref, w2b_ref):
    w1a_ref[...] = w1_ref[0:w1a_ref.shape[0], :].astype(jnp.bfloat16)
    w2b_ref[...] = w2_ref[...].astype(jnp.bfloat16)


def _mlp_body(emb_ref, small_ref, mask_ref,
              w1a_ref, w1b_ref, w2_ref, b1_ref, b2_ref, out_ref):
    x = emb_ref[...].astype(jnp.bfloat16)
    acc = jnp.dot(x, w1a_ref[...], preferred_element_type=jnp.float32)
    acc += jnp.dot(small_ref[...], w1b_ref[...],
                   preferred_element_type=jnp.float32)
    acc += b1_ref[...]
    h = jnp.maximum(acc, 0.0).astype(jnp.bfloat16)
    out = jnp.dot(h, w2_ref[...], preferred_element_type=jnp.float32)
    out += b2_ref[...]
    out_ref[...] = out * mask_ref[...]


def kernel(embeddings, visibility_scores, bbox_ltwh, keypoints_xyc,
           feats_masks, W1, b1, W2, b2):
    B, N, E = embeddings.shape
    M = B * N
    KPF = keypoints_xyc.shape[2] * keypoints_xyc.shape[3]
    F = W1.shape[1]
    T = W2.shape[1]
    S = F - E
    NB = N // _BM

    emb = embeddings.reshape(M // _BM, _BM, E)
    small = jnp.concatenate(
        [visibility_scores.reshape(M, 1),
         bbox_ltwh.reshape(M, bbox_ltwh.shape[-1]),
         keypoints_xyc.reshape(M, KPF)],
        axis=-1).astype(jnp.bfloat16)
    maskf = feats_masks.reshape(M, 1).astype(jnp.float32)
    W1b = W1[E:].astype(jnp.bfloat16)
    b1r = b1.reshape(1, F)
    b2r = b2.reshape(1, T)

    W1a, W2b = pl.pallas_call(
        _prep_body,
        out_shape=(
            jax.ShapeDtypeStruct((E, F), jnp.bfloat16),
            jax.ShapeDtypeStruct((F, T), jnp.bfloat16),
        ),
    )(W1, W2)

    out = pl.pallas_call(
        _mlp_body,
        grid=(M // _BM,),
        in_specs=[
            pl.BlockSpec((None, _BM, E), lambda i: (i, 0, 0)),
            pl.BlockSpec((_BM, S), lambda i: (i, 0)),
            pl.BlockSpec((_BM, 1), lambda i: (i, 0)),
            pl.BlockSpec((E, F), lambda i: (0, 0)),
            pl.BlockSpec((S, F), lambda i: (0, 0)),
            pl.BlockSpec((F, T), lambda i: (0, 0)),
            pl.BlockSpec((1, F), lambda i: (0, 0)),
            pl.BlockSpec((1, T), lambda i: (0, 0)),
        ],
        out_specs=pl.BlockSpec(
            (None, _BM, T), lambda i: (i // NB, i % NB, 0)),
        out_shape=jax.ShapeDtypeStruct((B, N, T), jnp.float32),
    )(emb, small, maskf, W1a, W1b, W2b, b1r, b2r)
    return out
